# Initial kernel scaffold; baseline (speedup 1.0000x reference)
import jax
import jax.numpy as jnp
from jax.experimental import pallas as pl
from jax.experimental.pallas import tpu as pltpu

NATM = 384
NOPS = 8
NS = 4
R_MAX = 0.4


def _body(natm_ref, ops_ref, w_ref, xc_ref, xr_ref, out_ref):
    s = pl.program_id(0)
    natm = natm_ref[s]
    rowi = jax.lax.broadcasted_iota(jnp.int32, (NATM, NATM), 0)
    coli = jax.lax.broadcasted_iota(jnp.int32, (NATM, NATM), 1)
    valid = (rowi < natm) & (coli < natm)

    def pair(cols, rows):
        ms = []
        d2 = None
        for c in range(3):
            d = rows[c] - cols[c]
            m = jnp.where(d > 0.5, d - 1.0, jnp.where(d < -0.5, d + 1.0, d))
            ms.append(m)
            d2 = m * m if d2 is None else d2 + m * m
        msk = (jnp.sqrt(d2) <= R_MAX) & valid
        zero = jnp.float32(0.0)
        return [jnp.sum(jnp.where(msk, ms[c] * ms[c], zero)) for c in range(3)]

    cols0 = [xc_ref[0, c] for c in range(3)]
    rows0 = [xr_ref[0, c] for c in range(3)]
    o0 = pair(cols0, rows0)

    loss = jnp.float32(0.0)
    for j in range(NOPS):
        op = [[ops_ref[s, j, a, b] for b in range(3)] for a in range(3)]
        c1 = [jnp.mod(op[a][0] * cols0[0] + op[a][1] * cols0[1]
                      + op[a][2] * cols0[2], 1.0) for a in range(3)]
        r1 = [jnp.mod(op[a][0] * rows0[0] + op[a][1] * rows0[1]
                      + op[a][2] * rows0[2], 1.0) for a in range(3)]
        o1 = pair(c1, r1)
        dn = ((o1[0] - o0[0]) ** 2 + (o1[1] - o0[1]) ** 2
              + (o1[2] - o0[2]) ** 2)
        loss = loss + w_ref[s, j] * jnp.sqrt(dn)

    @pl.when(s == 0)
    def _():
        out_ref[0, 0] = jnp.float32(0.0)
    out_ref[0, 0] += loss


def kernel(fracs, natms, oprss, noprs):
    natms = natms.reshape(-1).astype(jnp.int32)
    noprs = noprs.reshape(-1).astype(jnp.int32)
    cum_a = jnp.cumsum(natms)
    fa = cum_a - natms
    cum_o = jnp.cumsum(noprs)
    oa = cum_o - noprs

    frT = jnp.pad(fracs, ((0, NATM), (0, 0))).T
    Xs = jax.vmap(
        lambda st: jax.lax.dynamic_slice(frT, (0, st), (3, NATM)))(fa)
    Xc = Xs[:, :, :, None]
    Xr = Xs[:, :, None, :]

    jidx = jnp.arange(NOPS, dtype=jnp.int32)[None, :]
    opidx = jnp.clip(oa[:, None] + jidx, 0, oprss.shape[0] - 1)
    ops = oprss[opidx]
    w = jnp.where(
        jidx < noprs[:, None],
        1.0 / (jnp.maximum(noprs, 1)[:, None].astype(jnp.float32) * NS),
        0.0).astype(jnp.float32)

    out = pl.pallas_call(
        _body,
        grid=(NS,),
        in_specs=[
            pl.BlockSpec(memory_space=pltpu.SMEM),
            pl.BlockSpec(memory_space=pltpu.SMEM),
            pl.BlockSpec(memory_space=pltpu.SMEM),
            pl.BlockSpec((1, 3, NATM, 1), lambda s: (s, 0, 0, 0)),
            pl.BlockSpec((1, 3, 1, NATM), lambda s: (s, 0, 0, 0)),
        ],
        out_specs=pl.BlockSpec((1, 1), lambda s: (0, 0)),
        out_shape=jax.ShapeDtypeStruct((1, 1), jnp.float32),
    )(natms, ops, w, Xc, Xr)
    return out[0, 0]


# speedup vs baseline: 26.3113x; 26.3113x over previous
import jax
import jax.numpy as jnp
from jax.experimental import pallas as pl
from jax.experimental.pallas import tpu as pltpu

NATM = 384
NOPS = 8
NS = 4
R_MAX = 0.4


def _body(natm_ref, ops_ref, w_ref, xc_ref, xr_ref, out_ref):
    s = pl.program_id(0)
    natm = natm_ref[s]
    rowi = jax.lax.broadcasted_iota(jnp.int32, (NATM, NATM), 0)
    coli = jax.lax.broadcasted_iota(jnp.int32, (NATM, NATM), 1)
    valid = (rowi < natm) & (coli < natm)

    def pair(cols, rows):
        ms = []
        d2 = None
        for c in range(3):
            d = rows[c] - cols[c]
            m = jnp.where(d > 0.5, d - 1.0, jnp.where(d < -0.5, d + 1.0, d))
            ms.append(m)
            d2 = m * m if d2 is None else d2 + m * m
        msk = (jnp.sqrt(d2) <= R_MAX) & valid
        zero = jnp.float32(0.0)
        return [jnp.sum(jnp.where(msk, ms[c] * ms[c], zero)) for c in range(3)]

    cols0 = [xc_ref[0, c] for c in range(3)]
    rows0 = [xr_ref[0, c] for c in range(3)]
    o0 = pair(cols0, rows0)

    loss = jnp.float32(0.0)
    for j in range(NOPS):
        op = [[ops_ref[s, j, a, b] for b in range(3)] for a in range(3)]
        c1 = [jnp.mod(op[a][0] * cols0[0] + op[a][1] * cols0[1]
                      + op[a][2] * cols0[2], 1.0) for a in range(3)]
        r1 = [jnp.mod(op[a][0] * rows0[0] + op[a][1] * rows0[1]
                      + op[a][2] * rows0[2], 1.0) for a in range(3)]
        o1 = pair(c1, r1)
        dn = ((o1[0] - o0[0]) ** 2 + (o1[1] - o0[1]) ** 2
              + (o1[2] - o0[2]) ** 2)
        loss = loss + w_ref[s, j] * jnp.sqrt(dn)

    @pl.when(s == 0)
    def _():
        out_ref[0, 0] = jnp.float32(0.0)
    out_ref[0, 0] += loss


def kernel(fracs, natms, oprss, noprs):
    natms = natms.reshape(-1).astype(jnp.int32)
    noprs = noprs.reshape(-1).astype(jnp.int32)
    cum_a = jnp.cumsum(natms)
    fa = cum_a - natms
    cum_o = jnp.cumsum(noprs)
    oa = cum_o - noprs

    frT = jnp.pad(fracs, ((0, NATM), (0, 0))).T
    Xs = jax.vmap(
        lambda st: jax.lax.dynamic_slice(frT, (0, st), (3, NATM)))(fa)
    Xc = Xs[:, :, :, None]
    Xr = Xs[:, :, None, :]

    jidx = jnp.arange(NOPS, dtype=jnp.int32)[None, :]
    opidx = jnp.clip(oa[:, None] + jidx, 0, oprss.shape[0] - 1)
    ops = oprss[opidx]
    w = jnp.where(
        jidx < noprs[:, None],
        1.0 / (jnp.maximum(noprs, 1)[:, None].astype(jnp.float32) * NS),
        0.0).astype(jnp.float32)

    out = pl.pallas_call(
        _body,
        grid=(NS,),
        in_specs=[
            pl.BlockSpec(memory_space=pltpu.SMEM),
            pl.BlockSpec(memory_space=pltpu.SMEM),
            pl.BlockSpec(memory_space=pltpu.SMEM),
            pl.BlockSpec((1, 3, NATM, 1), lambda s: (s, 0, 0, 0)),
            pl.BlockSpec((1, 3, 1, NATM), lambda s: (s, 0, 0, 0)),
        ],
        out_specs=pl.BlockSpec(memory_space=pltpu.SMEM),
        out_shape=jax.ShapeDtypeStruct((1, 1), jnp.float32),
    )(natms, ops, w, Xc, Xr)
    return out[0, 0]
